# baseline (speedup 1.0000x reference)
import functools

import jax
import jax.numpy as jnp
from jax import lax
from jax.experimental import pallas as pl
from jax.experimental.pallas import tpu as pltpu
from jax.experimental.pallas import tpu_sc as plsc

B = 1024
N = 100000
D = 64
TEMP = 0.05
MOM = 0.2

NB = 2048
GRID = (N + NB - 1) // NB

NUM_CORES = 2
NUM_SUBCORES = 16
NUM_WORKERS = NUM_CORES * NUM_SUBCORES
ROWS_PER_WORKER = B // NUM_WORKERS
LANES = 16
VPR = D // LANES


def _tc_body(x_ref, tcol_ref, trow_ref, f_ref,
             out_ref, xn_ref, win_ref, nf_ref, xns_ref):
    j = pl.program_id(0)

    @pl.when(j == 0)
    def _():
        x = x_ref[...]
        ss = jnp.sum(x * x, axis=1, keepdims=True)
        xn = x * lax.rsqrt(jnp.maximum(ss, 1e-24))
        xn_ref[...] = xn
        xns_ref[...] = xn * (1.0 / TEMP)
        trow = trow_ref[...]
        for c in range(B // 128):
            tcol = tcol_ref[pl.ds(c * 128, 128), :]
            eq = tcol == trow
            jidx = lax.broadcasted_iota(jnp.int32, (128, B), 1)
            win_ref[pl.ds(c * 128, 128), :] = jnp.max(
                jnp.where(eq, jidx, -1), axis=1, keepdims=True)

    f = f_ref[...]
    out_ref[...] = lax.dot_general(
        xns_ref[...], f, (((1,), (1,)), ((), ())),
        preferred_element_type=jnp.float32)
    nf_ref[...] = f


_tc_call = pl.pallas_call(
    _tc_body,
    grid=(GRID,),
    in_specs=[
        pl.BlockSpec((B, D), lambda j: (0, 0)),
        pl.BlockSpec((B, 1), lambda j: (0, 0)),
        pl.BlockSpec((1, B), lambda j: (0, 0)),
        pl.BlockSpec((NB, D), lambda j: (j, 0)),
    ],
    out_specs=[
        pl.BlockSpec((B, NB), lambda j: (0, j)),
        pl.BlockSpec((B, D), lambda j: (0, 0)),
        pl.BlockSpec((B, 1), lambda j: (0, 0)),
        pl.BlockSpec((NB, D), lambda j: (j, 0)),
    ],
    out_shape=[
        jax.ShapeDtypeStruct((B, N), jnp.float32),
        jax.ShapeDtypeStruct((B, D), jnp.float32),
        jax.ShapeDtypeStruct((B, 1), jnp.int32),
        jax.ShapeDtypeStruct((N, D), jnp.float32),
    ],
    scratch_shapes=[pltpu.VMEM((B, D), jnp.float32)],
    compiler_params=pltpu.CompilerParams(
        dimension_semantics=("arbitrary",),
        vmem_limit_bytes=100 * 1024 * 1024,
    ),
)


def _rsqrt_newton(t16):
    i = lax.bitcast_convert_type(t16, jnp.int32)
    y = lax.bitcast_convert_type(jnp.int32(0x5F3759DF) - (i >> 1), jnp.float32)
    for _ in range(3):
        y = y * (1.5 - 0.5 * t16 * y * y)
    return y


def _sc_body(feat_hbm, xn_hbm, tgt_hbm, win_hbm, new_hbm,
             tgt_v, win_v, g_v, xw_v, upd_v, sem1, sem2):
    c = lax.axis_index("c")
    s = lax.axis_index("s")
    wid = s * NUM_CORES + c
    base = wid * ROWS_PER_WORKER
    pltpu.sync_copy(tgt_hbm.at[pl.ds(base, ROWS_PER_WORKER)], tgt_v)
    pltpu.sync_copy(win_hbm.at[pl.ds(base, ROWS_PER_WORKER)], win_v)
    tvecs = [tgt_v[pl.ds(q * LANES, LANES)]
             for q in range(ROWS_PER_WORKER // LANES)]
    wvecs = [win_v[pl.ds(q * LANES, LANES)]
             for q in range(ROWS_PER_WORKER // LANES)]
    copies = []
    for r in range(ROWS_PER_WORKER):
        t_s = tvecs[r // LANES][r % LANES]
        w_s = wvecs[r // LANES][r % LANES]
        copies.append(pltpu.async_copy(
            feat_hbm.at[pl.ds(t_s, 1), :], g_v.at[pl.ds(r, 1), :], sem1))
        copies.append(pltpu.async_copy(
            xn_hbm.at[pl.ds(w_s, 1), :], xw_v.at[pl.ds(r, 1), :], sem2))
    for cp in copies:
        cp.wait()
    for r in range(ROWS_PER_WORKER):
        us = []
        ssum = None
        for k in range(VPR):
            g = g_v[r, pl.ds(k * LANES, LANES)]
            w = xw_v[r, pl.ds(k * LANES, LANES)]
            u = MOM * g + (1.0 - MOM) * w
            us.append(u)
            p = u * u
            ssum = p if ssum is None else ssum + p
        tot = jnp.sum(ssum)
        t16 = jnp.full((LANES,), tot, jnp.float32)
        y = _rsqrt_newton(t16)
        for k in range(VPR):
            upd_v[r, pl.ds(k * LANES, LANES)] = us[k] * y
    scat = []
    for r in range(ROWS_PER_WORKER):
        t_s = tvecs[r // LANES][r % LANES]
        scat.append(pltpu.async_copy(
            upd_v.at[pl.ds(r, 1), :], new_hbm.at[pl.ds(t_s, 1), :], sem1))
    for cp in scat:
        cp.wait()


_sc_update = pl.kernel(
    _sc_body,
    out_type=(),
    mesh=plsc.VectorSubcoreMesh(
        core_axis_name="c", subcore_axis_name="s",
        num_cores=NUM_CORES, num_subcores=NUM_SUBCORES),
    compiler_params=pltpu.CompilerParams(needs_layout_passes=False),
    scratch_types=[
        pltpu.VMEM((ROWS_PER_WORKER,), jnp.int32),
        pltpu.VMEM((ROWS_PER_WORKER,), jnp.int32),
        pltpu.VMEM((ROWS_PER_WORKER, D), jnp.float32),
        pltpu.VMEM((ROWS_PER_WORKER, D), jnp.float32),
        pltpu.VMEM((ROWS_PER_WORKER, D), jnp.float32),
        pltpu.SemaphoreType.DMA,
        pltpu.SemaphoreType.DMA,
    ],
)


def kernel(inputs, inputs_logits, targets, indexes, features):
    del inputs_logits, indexes
    t = targets.astype(jnp.int32)
    t_col = t.reshape(B, 1)
    t_row = t.reshape(1, B)
    outputs, xn, winner, new_f = _tc_call(inputs, t_col, t_row, features)
    new_ref = jax.new_ref(new_f)
    _sc_update(features, xn, t, winner.reshape(B), new_ref)
    return outputs, new_ref[...]
